# baseline (speedup 1.0000x reference)
import jax
import jax.numpy as jnp
from jax.experimental import pallas as pl
from jax.experimental.pallas import tpu as pltpu

_EPS = 1e-5
_BF = jnp.bfloat16


def _body(x_ref,
          w_inc1, s_inc1, b_inc1, w_inc2, s_inc2, b_inc2,
          w_d0c0, s_d0c0, b_d0c0, w_d0c1, s_d0c1, b_d0c1,
          w_d0c2, s_d0c2, b_d0c2,
          w_d1c0, s_d1c0, b_d1c0, w_d1c1, s_d1c1, b_d1c1,
          w_d1c2, s_d1c2, b_d1c2,
          w_u2c1, s_u2c1, b_u2c1, w_u2c2, s_u2c2, b_u2c2, w_u2c3,
          w_u1c1, s_u1c1, b_u1c1, w_u1c2, s_u1c2, b_u1c2, w_u1c3,
          w_oc1, s_oc1, b_oc1, w_oc2,
          o_ref,
          pad12, pad64, pad128, pads32, pads64, up2, up1):

    def conv(act, pad, w_ref, sb, relu, stride=1):
        h2, w2, cin = pad.shape
        hi, wi = h2 - 2, w2 - 2
        pad[...] = jnp.zeros_like(pad)
        pad[1:hi + 1, 1:wi + 1, :] = act.astype(pad.dtype)
        ho, wo = hi // stride, wi // stride
        taps = [pad[pl.ds(dy, ho, stride), pl.ds(dx, wo, stride), :]
                for dy in range(3) for dx in range(3)]
        slab = jnp.concatenate(taps, axis=-1).reshape(ho * wo, 9 * cin)
        y = jnp.dot(slab.astype(_BF), w_ref[...],
                    preferred_element_type=jnp.float32)
        if sb is not None:
            y = y * sb[0][...] + sb[1][...]
        if relu:
            y = jnp.maximum(y, 0.0)
        return y.reshape(ho, wo, y.shape[-1])

    def conv_z(act, w9_ref, sb, relu, cout):
        hh, ww, cin = act.shape
        z = jnp.dot(act.reshape(hh * ww, cin).astype(_BF), w9_ref[...],
                    preferred_element_type=jnp.float32).reshape(hh, ww,
                                                                9 * cout)
        zero_col = jnp.zeros((hh, 1, cout), jnp.float32)
        zero_row = jnp.zeros((1, ww, cout), jnp.float32)
        y = None
        for dy in range(3):
            for dx in range(3):
                t = dy * 3 + dx
                zt = z[:, :, t * cout:(t + 1) * cout]
                if dx == 0:
                    zt = jnp.concatenate([zero_col, zt[:, :ww - 1, :]], axis=1)
                elif dx == 2:
                    zt = jnp.concatenate([zt[:, 1:, :], zero_col], axis=1)
                if dy == 0:
                    zt = jnp.concatenate([zero_row, zt[:hh - 1, :, :]], axis=0)
                elif dy == 2:
                    zt = jnp.concatenate([zt[1:, :, :], zero_row], axis=0)
                y = zt if y is None else y + zt
        if sb is not None:
            y = y * sb[0][...] + sb[1][...]
        if relu:
            y = jnp.maximum(y, 0.0)
        return y

    def shuffle(y, up_ref):
        hq, wq, c4 = y.shape
        c = c4 // 4
        for r1 in range(2):
            for r2 in range(2):
                q = 2 * r1 + r2
                up_ref[pl.ds(r1, hq, 2), pl.ds(r2, wq, 2), :] = (
                    y[:, :, q * c:(q + 1) * c])

    x = x_ref[0]
    x0 = conv(x, pad12, w_inc1, (s_inc1, b_inc1), True)
    x0 = conv_z(x0, w_inc2, (s_inc2, b_inc2), True, 32)
    t = conv(x0, pads32, w_d0c0, (s_d0c0, b_d0c0), True, stride=2)
    t = conv(t, pad64, w_d0c1, (s_d0c1, b_d0c1), True)
    x1 = conv(t, pad64, w_d0c2, (s_d0c2, b_d0c2), True)
    t = conv(x1, pads64, w_d1c0, (s_d1c0, b_d1c0), True, stride=2)
    t = conv(t, pad128, w_d1c1, (s_d1c1, b_d1c1), True)
    t = conv(t, pad128, w_d1c2, (s_d1c2, b_d1c2), True)
    t = conv(t, pad128, w_u2c1, (s_u2c1, b_u2c1), True)
    t = conv(t, pad128, w_u2c2, (s_u2c2, b_u2c2), True)
    t = conv(t, pad128, w_u2c3, None, False)
    shuffle(t, up2)
    t = x1 + up2[...]
    t = conv(t, pad64, w_u1c1, (s_u1c1, b_u1c1), True)
    t = conv(t, pad64, w_u1c2, (s_u1c2, b_u1c2), True)
    t = conv(t, pad64, w_u1c3, None, False)
    shuffle(t, up1)
    t = x0 + up1[...]
    t = conv_z(t, w_oc1, (s_oc1, b_oc1), True, 32)
    y = conv_z(t, w_oc2, None, False, 3)
    o_ref[...] = y[None]


def _affine(gamma, beta, mean, var):
    s = gamma / jnp.sqrt(var + _EPS)
    return s[None, :].astype(jnp.float32), (beta - mean * s)[None, :].astype(
        jnp.float32)


def _flat(w):
    return w.reshape(9 * w.shape[2], w.shape[3]).astype(_BF)


def _flat_z(w):
    cin, cout = w.shape[2], w.shape[3]
    return (w.reshape(9, cin, cout).transpose(1, 0, 2)
            .reshape(cin, 9 * cout).astype(_BF))


def _flat_shuffled(w):
    k, cout = 9 * w.shape[2], w.shape[3]
    wf = w.reshape(k, cout)
    return (wf.reshape(k, cout // 4, 2, 2).transpose(0, 2, 3, 1)
            .reshape(k, cout).astype(_BF))


def _block_diag_grouped(w, groups):
    kh, kw, cin_g, cout = w.shape
    cin, cout_g = cin_g * groups, cout // groups
    wd = jnp.zeros((kh, kw, cin, cout), w.dtype)
    for g in range(groups):
        wd = wd.at[:, :, g * cin_g:(g + 1) * cin_g,
                   g * cout_g:(g + 1) * cout_g].set(
                       w[:, :, :, g * cout_g:(g + 1) * cout_g])
    return wd


def kernel(in0, in1, in2, noise_map,
           inc1_w, inc1_gamma, inc1_beta, inc1_mean, inc1_var,
           inc2_w, inc2_gamma, inc2_beta, inc2_mean, inc2_var,
           d0_c0_w, d0_c0_gamma, d0_c0_beta, d0_c0_mean, d0_c0_var,
           d0_c1_w, d0_c1_gamma, d0_c1_beta, d0_c1_mean, d0_c1_var,
           d0_c2_w, d0_c2_gamma, d0_c2_beta, d0_c2_mean, d0_c2_var,
           d1_c0_w, d1_c0_gamma, d1_c0_beta, d1_c0_mean, d1_c0_var,
           d1_c1_w, d1_c1_gamma, d1_c1_beta, d1_c1_mean, d1_c1_var,
           d1_c2_w, d1_c2_gamma, d1_c2_beta, d1_c2_mean, d1_c2_var,
           u2_c1_w, u2_c1_gamma, u2_c1_beta, u2_c1_mean, u2_c1_var,
           u2_c2_w, u2_c2_gamma, u2_c2_beta, u2_c2_mean, u2_c2_var,
           u2_c3_w,
           u1_c1_w, u1_c1_gamma, u1_c1_beta, u1_c1_mean, u1_c1_var,
           u1_c2_w, u1_c2_gamma, u1_c2_beta, u1_c2_mean, u1_c2_var,
           u1_c3_w,
           o_c1_w, o_c1_gamma, o_c1_beta, o_c1_mean, o_c1_var,
           o_c2_w):
    B, _, H, W = in0.shape
    nh = lambda t: jnp.transpose(t, (0, 2, 3, 1))
    nm = nh(noise_map)
    x_in = jnp.concatenate(
        [nh(in0), nm, nh(in1), nm, nh(in2), nm], axis=-1).astype(_BF)

    args = [x_in,
            _flat(_block_diag_grouped(inc1_w, 3)),
            *_affine(inc1_gamma, inc1_beta, inc1_mean, inc1_var),
            _flat_z(inc2_w),
            *_affine(inc2_gamma, inc2_beta, inc2_mean, inc2_var),
            _flat(d0_c0_w),
            *_affine(d0_c0_gamma, d0_c0_beta, d0_c0_mean, d0_c0_var),
            _flat(d0_c1_w),
            *_affine(d0_c1_gamma, d0_c1_beta, d0_c1_mean, d0_c1_var),
            _flat(d0_c2_w),
            *_affine(d0_c2_gamma, d0_c2_beta, d0_c2_mean, d0_c2_var),
            _flat(d1_c0_w),
            *_affine(d1_c0_gamma, d1_c0_beta, d1_c0_mean, d1_c0_var),
            _flat(d1_c1_w),
            *_affine(d1_c1_gamma, d1_c1_beta, d1_c1_mean, d1_c1_var),
            _flat(d1_c2_w),
            *_affine(d1_c2_gamma, d1_c2_beta, d1_c2_mean, d1_c2_var),
            _flat(u2_c1_w),
            *_affine(u2_c1_gamma, u2_c1_beta, u2_c1_mean, u2_c1_var),
            _flat(u2_c2_w),
            *_affine(u2_c2_gamma, u2_c2_beta, u2_c2_mean, u2_c2_var),
            _flat_shuffled(u2_c3_w),
            _flat(u1_c1_w),
            *_affine(u1_c1_gamma, u1_c1_beta, u1_c1_mean, u1_c1_var),
            _flat(u1_c2_w),
            *_affine(u1_c2_gamma, u1_c2_beta, u1_c2_mean, u1_c2_var),
            _flat_shuffled(u1_c3_w),
            _flat_z(o_c1_w),
            *_affine(o_c1_gamma, o_c1_beta, o_c1_mean, o_c1_var),
            _flat_z(o_c2_w)]

    in_specs = [pl.BlockSpec((1, H, W, 12), lambda b: (b, 0, 0, 0))]
    for a in args[1:]:
        nd = a.ndim
        in_specs.append(pl.BlockSpec(a.shape, lambda b, _n=nd: (0,) * _n))

    y = pl.pallas_call(
        _body,
        out_shape=jax.ShapeDtypeStruct((B, H, W, 3), jnp.float32),
        grid_spec=pltpu.PrefetchScalarGridSpec(
            num_scalar_prefetch=0,
            grid=(B,),
            in_specs=in_specs,
            out_specs=pl.BlockSpec((1, H, W, 3), lambda b: (b, 0, 0, 0)),
            scratch_shapes=[
                pltpu.VMEM((H + 2, W + 2, 12), _BF),
                pltpu.VMEM((H // 2 + 2, W // 2 + 2, 64), _BF),
                pltpu.VMEM((H // 4 + 2, W // 4 + 2, 128), _BF),
                pltpu.VMEM((H + 2, W + 2, 32), jnp.float32),
                pltpu.VMEM((H // 2 + 2, W // 2 + 2, 64), jnp.float32),
                pltpu.VMEM((H // 2, W // 2, 64), jnp.float32),
                pltpu.VMEM((H, W, 32), jnp.float32),
            ]),
        compiler_params=pltpu.CompilerParams(
            dimension_semantics=("parallel",),
            vmem_limit_bytes=100 * 1024 * 1024),
    )(*args)

    return jnp.transpose(nh(in1) - y, (0, 3, 1, 2))


# speedup vs baseline: 1.1917x; 1.1917x over previous
import jax
import jax.numpy as jnp
from jax.experimental import pallas as pl
from jax.experimental.pallas import tpu as pltpu

_EPS = 1e-5
_BF = jnp.bfloat16


def _body(x_ref,
          w_inc1, s_inc1, b_inc1, w_inc2, s_inc2, b_inc2,
          w_d0c0, s_d0c0, b_d0c0, w_d0c1, s_d0c1, b_d0c1,
          w_d0c2, s_d0c2, b_d0c2,
          w_d1c0, s_d1c0, b_d1c0, w_d1c1, s_d1c1, b_d1c1,
          w_d1c2, s_d1c2, b_d1c2,
          w_u2c1, s_u2c1, b_u2c1, w_u2c2, s_u2c2, b_u2c2, w_u2c3,
          w_u1c1, s_u1c1, b_u1c1, w_u1c2, s_u1c2, b_u1c2, w_u1c3,
          w_oc1, s_oc1, b_oc1, w_oc2,
          o_ref,
          sh12, sh90, sh32, sh64, sh128, pads32, pads64, up2, up1):

    def conv(act, sh, w_ref, sb, relu):
        _, h2, w, cin = sh.shape
        hi = h2 - 2
        a = act.astype(_BF)
        zrow = jnp.zeros((1, w, cin), _BF)
        zcol = jnp.zeros((hi, 1, cin), _BF)
        sh[1, 1:hi + 1, :, :] = a
        sh[0, 1:hi + 1, 1:, :] = a[:, :w - 1, :]
        sh[2, 1:hi + 1, :w - 1, :] = a[:, 1:, :]
        for k in range(3):
            sh[k, 0:1, :, :] = zrow
            sh[k, hi + 1:hi + 2, :, :] = zrow
        sh[0, 1:hi + 1, 0:1, :] = zcol
        sh[2, 1:hi + 1, w - 1:w, :] = zcol
        taps = [sh[dx, pl.ds(dy, hi), :, :]
                for dy in range(3) for dx in range(3)]
        slab = jnp.concatenate(taps, axis=-1).reshape(hi * w, 9 * cin)
        y = jnp.dot(slab, w_ref[...], preferred_element_type=jnp.float32)
        if sb is not None:
            y = y * sb[0][...] + sb[1][...]
        if relu:
            y = jnp.maximum(y, 0.0)
        return y.reshape(hi, w, y.shape[-1])

    def conv_s2(act, pad, w_ref, sb, relu):
        h2, w2, cin = pad.shape
        hi, wi = h2 - 2, w2 - 2
        pad[...] = jnp.zeros_like(pad)
        pad[1:hi + 1, 1:wi + 1, :] = act.astype(pad.dtype)
        ho, wo = hi // 2, wi // 2
        taps = [pad[pl.ds(dy, ho, 2), pl.ds(dx, wo, 2), :]
                for dy in range(3) for dx in range(3)]
        slab = jnp.concatenate(taps, axis=-1).reshape(ho * wo, 9 * cin)
        y = jnp.dot(slab.astype(_BF), w_ref[...],
                    preferred_element_type=jnp.float32)
        if sb is not None:
            y = y * sb[0][...] + sb[1][...]
        if relu:
            y = jnp.maximum(y, 0.0)
        return y.reshape(ho, wo, y.shape[-1])

    def shuffle(y, up_ref):
        hq, wq, c4 = y.shape
        c = c4 // 4
        for r1 in range(2):
            for r2 in range(2):
                q = 2 * r1 + r2
                up_ref[pl.ds(r1, hq, 2), pl.ds(r2, wq, 2), :] = (
                    y[:, :, q * c:(q + 1) * c])

    x = x_ref[0]
    x0 = conv(x, sh12, w_inc1, (s_inc1, b_inc1), True)
    x0 = conv(x0, sh90, w_inc2, (s_inc2, b_inc2), True)
    t = conv_s2(x0, pads32, w_d0c0, (s_d0c0, b_d0c0), True)
    t = conv(t, sh64, w_d0c1, (s_d0c1, b_d0c1), True)
    x1 = conv(t, sh64, w_d0c2, (s_d0c2, b_d0c2), True)
    t = conv_s2(x1, pads64, w_d1c0, (s_d1c0, b_d1c0), True)
    t = conv(t, sh128, w_d1c1, (s_d1c1, b_d1c1), True)
    t = conv(t, sh128, w_d1c2, (s_d1c2, b_d1c2), True)
    t = conv(t, sh128, w_u2c1, (s_u2c1, b_u2c1), True)
    t = conv(t, sh128, w_u2c2, (s_u2c2, b_u2c2), True)
    t = conv(t, sh128, w_u2c3, None, False)
    shuffle(t, up2)
    t = x1 + up2[...]
    t = conv(t, sh64, w_u1c1, (s_u1c1, b_u1c1), True)
    t = conv(t, sh64, w_u1c2, (s_u1c2, b_u1c2), True)
    t = conv(t, sh64, w_u1c3, None, False)
    shuffle(t, up1)
    t = x0 + up1[...]
    t = conv(t, sh32, w_oc1, (s_oc1, b_oc1), True)
    y = conv(t, sh32, w_oc2, None, False)
    o_ref[...] = y[None]


def _affine(gamma, beta, mean, var):
    s = gamma / jnp.sqrt(var + _EPS)
    return s[None, :].astype(jnp.float32), (beta - mean * s)[None, :].astype(
        jnp.float32)


def _flat(w):
    return w.reshape(9 * w.shape[2], w.shape[3]).astype(_BF)


def _flat_z(w):
    cin, cout = w.shape[2], w.shape[3]
    return (w.reshape(9, cin, cout).transpose(1, 0, 2)
            .reshape(cin, 9 * cout).astype(_BF))


def _flat_shuffled(w):
    k, cout = 9 * w.shape[2], w.shape[3]
    wf = w.reshape(k, cout)
    return (wf.reshape(k, cout // 4, 2, 2).transpose(0, 2, 3, 1)
            .reshape(k, cout).astype(_BF))


def _block_diag_grouped(w, groups):
    kh, kw, cin_g, cout = w.shape
    cin, cout_g = cin_g * groups, cout // groups
    wd = jnp.zeros((kh, kw, cin, cout), w.dtype)
    for g in range(groups):
        wd = wd.at[:, :, g * cin_g:(g + 1) * cin_g,
                   g * cout_g:(g + 1) * cout_g].set(
                       w[:, :, :, g * cout_g:(g + 1) * cout_g])
    return wd


def kernel(in0, in1, in2, noise_map,
           inc1_w, inc1_gamma, inc1_beta, inc1_mean, inc1_var,
           inc2_w, inc2_gamma, inc2_beta, inc2_mean, inc2_var,
           d0_c0_w, d0_c0_gamma, d0_c0_beta, d0_c0_mean, d0_c0_var,
           d0_c1_w, d0_c1_gamma, d0_c1_beta, d0_c1_mean, d0_c1_var,
           d0_c2_w, d0_c2_gamma, d0_c2_beta, d0_c2_mean, d0_c2_var,
           d1_c0_w, d1_c0_gamma, d1_c0_beta, d1_c0_mean, d1_c0_var,
           d1_c1_w, d1_c1_gamma, d1_c1_beta, d1_c1_mean, d1_c1_var,
           d1_c2_w, d1_c2_gamma, d1_c2_beta, d1_c2_mean, d1_c2_var,
           u2_c1_w, u2_c1_gamma, u2_c1_beta, u2_c1_mean, u2_c1_var,
           u2_c2_w, u2_c2_gamma, u2_c2_beta, u2_c2_mean, u2_c2_var,
           u2_c3_w,
           u1_c1_w, u1_c1_gamma, u1_c1_beta, u1_c1_mean, u1_c1_var,
           u1_c2_w, u1_c2_gamma, u1_c2_beta, u1_c2_mean, u1_c2_var,
           u1_c3_w,
           o_c1_w, o_c1_gamma, o_c1_beta, o_c1_mean, o_c1_var,
           o_c2_w):
    B, _, H, W = in0.shape
    nh = lambda t: jnp.transpose(t, (0, 2, 3, 1))
    nm = nh(noise_map)
    x_in = jnp.concatenate(
        [nh(in0), nm, nh(in1), nm, nh(in2), nm], axis=-1).astype(_BF)

    args = [x_in,
            _flat(_block_diag_grouped(inc1_w, 3)),
            *_affine(inc1_gamma, inc1_beta, inc1_mean, inc1_var),
            _flat(inc2_w), *_affine(inc2_gamma, inc2_beta, inc2_mean, inc2_var),
            _flat(d0_c0_w),
            *_affine(d0_c0_gamma, d0_c0_beta, d0_c0_mean, d0_c0_var),
            _flat(d0_c1_w),
            *_affine(d0_c1_gamma, d0_c1_beta, d0_c1_mean, d0_c1_var),
            _flat(d0_c2_w),
            *_affine(d0_c2_gamma, d0_c2_beta, d0_c2_mean, d0_c2_var),
            _flat(d1_c0_w),
            *_affine(d1_c0_gamma, d1_c0_beta, d1_c0_mean, d1_c0_var),
            _flat(d1_c1_w),
            *_affine(d1_c1_gamma, d1_c1_beta, d1_c1_mean, d1_c1_var),
            _flat(d1_c2_w),
            *_affine(d1_c2_gamma, d1_c2_beta, d1_c2_mean, d1_c2_var),
            _flat(u2_c1_w),
            *_affine(u2_c1_gamma, u2_c1_beta, u2_c1_mean, u2_c1_var),
            _flat(u2_c2_w),
            *_affine(u2_c2_gamma, u2_c2_beta, u2_c2_mean, u2_c2_var),
            _flat_shuffled(u2_c3_w),
            _flat(u1_c1_w),
            *_affine(u1_c1_gamma, u1_c1_beta, u1_c1_mean, u1_c1_var),
            _flat(u1_c2_w),
            *_affine(u1_c2_gamma, u1_c2_beta, u1_c2_mean, u1_c2_var),
            _flat_shuffled(u1_c3_w),
            _flat(o_c1_w),
            *_affine(o_c1_gamma, o_c1_beta, o_c1_mean, o_c1_var),
            _flat(o_c2_w)]

    in_specs = [pl.BlockSpec((1, H, W, 12), lambda b: (b, 0, 0, 0))]
    for a in args[1:]:
        nd = a.ndim
        in_specs.append(pl.BlockSpec(a.shape, lambda b, _n=nd: (0,) * _n))

    y = pl.pallas_call(
        _body,
        out_shape=jax.ShapeDtypeStruct((B, H, W, 3), jnp.float32),
        grid_spec=pltpu.PrefetchScalarGridSpec(
            num_scalar_prefetch=0,
            grid=(B,),
            in_specs=in_specs,
            out_specs=pl.BlockSpec((1, H, W, 3), lambda b: (b, 0, 0, 0)),
            scratch_shapes=[
                pltpu.VMEM((3, H + 2, W, 12), _BF),
                pltpu.VMEM((3, H + 2, W, 90), _BF),
                pltpu.VMEM((3, H + 2, W, 32), _BF),
                pltpu.VMEM((3, H // 2 + 2, W // 2, 64), _BF),
                pltpu.VMEM((3, H // 4 + 2, W // 4, 128), _BF),
                pltpu.VMEM((H + 2, W + 2, 32), jnp.float32),
                pltpu.VMEM((H // 2 + 2, W // 2 + 2, 64), jnp.float32),
                pltpu.VMEM((H // 2, W // 2, 64), jnp.float32),
                pltpu.VMEM((H, W, 32), jnp.float32),
            ]),
        compiler_params=pltpu.CompilerParams(
            dimension_semantics=("parallel",),
            vmem_limit_bytes=100 * 1024 * 1024),
    )(*args)

    return jnp.transpose(nh(in1) - y, (0, 3, 1, 2))


# speedup vs baseline: 1.2722x; 1.0675x over previous
import jax
import jax.numpy as jnp
from jax.experimental import pallas as pl
from jax.experimental.pallas import tpu as pltpu

_EPS = 1e-5
_BF = jnp.bfloat16


def _body(x_ref,
          w_inc1, s_inc1, b_inc1, w_inc2, s_inc2, b_inc2,
          w_d0c0, s_d0c0, b_d0c0, w_d0c1, s_d0c1, b_d0c1,
          w_d0c2, s_d0c2, b_d0c2,
          w_d1c0, s_d1c0, b_d1c0, w_d1c1, s_d1c1, b_d1c1,
          w_d1c2, s_d1c2, b_d1c2,
          w_u2c1, s_u2c1, b_u2c1, w_u2c2, s_u2c2, b_u2c2, w_u2c3,
          w_u1c1, s_u1c1, b_u1c1, w_u1c2, s_u1c2, b_u1c2, w_u1c3,
          w_oc1, s_oc1, b_oc1, w_oc2,
          o_ref,
          sh12, sh90, sh32, sh64, sh128, pads32, pads64, up2, up1):

    def conv(act, sh, w_ref, sb, relu):
        h2, w, c3 = sh.shape
        hi = h2 - 2
        cin = act.shape[-1]
        a = act.astype(_BF)
        zc = jnp.zeros((hi, 1, cin), _BF)
        pieces = [jnp.concatenate([zc, a[:, :w - 1, :]], axis=1), a,
                  jnp.concatenate([a[:, 1:, :], zc], axis=1)]
        if c3 > 3 * cin:
            pieces.append(jnp.zeros((hi, w, c3 - 3 * cin), _BF))
        zrow = jnp.zeros((1, w, c3), _BF)
        sh[0:1, :, :] = zrow
        sh[h2 - 1:h2, :, :] = zrow
        sh[1:hi + 1, :, :] = jnp.concatenate(pieces, axis=-1)
        slab = jnp.concatenate([sh[pl.ds(dy, hi), :, :] for dy in range(3)],
                               axis=-1).reshape(hi * w, 3 * c3)
        y = jnp.dot(slab, w_ref[...], preferred_element_type=jnp.float32)
        if sb is not None:
            y = y * sb[0][...] + sb[1][...]
        if relu:
            y = jnp.maximum(y, 0.0)
        return y.reshape(hi, w, y.shape[-1])

    def conv_s2(act, pad, w_ref, sb, relu):
        h2, w2, cin = pad.shape
        hi, wi = h2 - 2, w2 - 2
        pad[...] = jnp.zeros_like(pad)
        pad[1:hi + 1, 1:wi + 1, :] = act.astype(pad.dtype)
        ho, wo = hi // 2, wi // 2
        taps = [pad[pl.ds(dy, ho, 2), pl.ds(dx, wo, 2), :]
                for dy in range(3) for dx in range(3)]
        slab = jnp.concatenate(taps, axis=-1).reshape(ho * wo, 9 * cin)
        y = jnp.dot(slab.astype(_BF), w_ref[...],
                    preferred_element_type=jnp.float32)
        if sb is not None:
            y = y * sb[0][...] + sb[1][...]
        if relu:
            y = jnp.maximum(y, 0.0)
        return y.reshape(ho, wo, y.shape[-1])

    def shuffle(y, up_ref):
        hq, wq, c4 = y.shape
        c = c4 // 4
        for r1 in range(2):
            for r2 in range(2):
                q = 2 * r1 + r2
                up_ref[pl.ds(r1, hq, 2), pl.ds(r2, wq, 2), :] = (
                    y[:, :, q * c:(q + 1) * c])

    x = x_ref[0]
    x0 = conv(x, sh12, w_inc1, (s_inc1, b_inc1), True)
    x0 = conv(x0, sh90, w_inc2, (s_inc2, b_inc2), True)
    t = conv_s2(x0, pads32, w_d0c0, (s_d0c0, b_d0c0), True)
    t = conv(t, sh64, w_d0c1, (s_d0c1, b_d0c1), True)
    x1 = conv(t, sh64, w_d0c2, (s_d0c2, b_d0c2), True)
    t = conv_s2(x1, pads64, w_d1c0, (s_d1c0, b_d1c0), True)
    t = conv(t, sh128, w_d1c1, (s_d1c1, b_d1c1), True)
    t = conv(t, sh128, w_d1c2, (s_d1c2, b_d1c2), True)
    t = conv(t, sh128, w_u2c1, (s_u2c1, b_u2c1), True)
    t = conv(t, sh128, w_u2c2, (s_u2c2, b_u2c2), True)
    t = conv(t, sh128, w_u2c3, None, False)
    shuffle(t, up2)
    t = x1 + up2[...]
    t = conv(t, sh64, w_u1c1, (s_u1c1, b_u1c1), True)
    t = conv(t, sh64, w_u1c2, (s_u1c2, b_u1c2), True)
    t = conv(t, sh64, w_u1c3, None, False)
    shuffle(t, up1)
    t = x0 + up1[...]
    t = conv(t, sh32, w_oc1, (s_oc1, b_oc1), True)
    y = conv(t, sh32, w_oc2, None, False)
    o_ref[...] = y[None]


def _affine(gamma, beta, mean, var):
    s = gamma / jnp.sqrt(var + _EPS)
    return s[None, :].astype(jnp.float32), (beta - mean * s)[None, :].astype(
        jnp.float32)


def _flat(w):
    return w.reshape(9 * w.shape[2], w.shape[3]).astype(_BF)


def _flat_lane(w, c3):
    cin, cout = w.shape[2], w.shape[3]
    wf = w.reshape(3, 3 * cin, cout)
    wp = jnp.zeros((3, c3, cout), w.dtype).at[:, :3 * cin, :].set(wf)
    return wp.reshape(3 * c3, cout).astype(_BF)


def _shuffle_cols(w):
    kh, kw, cin, cout = w.shape
    return (w.reshape(kh, kw, cin, cout // 4, 2, 2)
            .transpose(0, 1, 2, 4, 5, 3).reshape(kh, kw, cin, cout))


def _block_diag_grouped(w, groups):
    kh, kw, cin_g, cout = w.shape
    cin, cout_g = cin_g * groups, cout // groups
    wd = jnp.zeros((kh, kw, cin, cout), w.dtype)
    for g in range(groups):
        wd = wd.at[:, :, g * cin_g:(g + 1) * cin_g,
                   g * cout_g:(g + 1) * cout_g].set(
                       w[:, :, :, g * cout_g:(g + 1) * cout_g])
    return wd


def kernel(in0, in1, in2, noise_map,
           inc1_w, inc1_gamma, inc1_beta, inc1_mean, inc1_var,
           inc2_w, inc2_gamma, inc2_beta, inc2_mean, inc2_var,
           d0_c0_w, d0_c0_gamma, d0_c0_beta, d0_c0_mean, d0_c0_var,
           d0_c1_w, d0_c1_gamma, d0_c1_beta, d0_c1_mean, d0_c1_var,
           d0_c2_w, d0_c2_gamma, d0_c2_beta, d0_c2_mean, d0_c2_var,
           d1_c0_w, d1_c0_gamma, d1_c0_beta, d1_c0_mean, d1_c0_var,
           d1_c1_w, d1_c1_gamma, d1_c1_beta, d1_c1_mean, d1_c1_var,
           d1_c2_w, d1_c2_gamma, d1_c2_beta, d1_c2_mean, d1_c2_var,
           u2_c1_w, u2_c1_gamma, u2_c1_beta, u2_c1_mean, u2_c1_var,
           u2_c2_w, u2_c2_gamma, u2_c2_beta, u2_c2_mean, u2_c2_var,
           u2_c3_w,
           u1_c1_w, u1_c1_gamma, u1_c1_beta, u1_c1_mean, u1_c1_var,
           u1_c2_w, u1_c2_gamma, u1_c2_beta, u1_c2_mean, u1_c2_var,
           u1_c3_w,
           o_c1_w, o_c1_gamma, o_c1_beta, o_c1_mean, o_c1_var,
           o_c2_w):
    B, _, H, W = in0.shape
    nh = lambda t: jnp.transpose(t, (0, 2, 3, 1))
    nm = nh(noise_map)
    x_in = jnp.concatenate(
        [nh(in0), nm, nh(in1), nm, nh(in2), nm], axis=-1).astype(_BF)

    args = [x_in,
            _flat_lane(_block_diag_grouped(inc1_w, 3), 128),
            *_affine(inc1_gamma, inc1_beta, inc1_mean, inc1_var),
            _flat_lane(inc2_w, 384),
            *_affine(inc2_gamma, inc2_beta, inc2_mean, inc2_var),
            _flat(d0_c0_w),
            *_affine(d0_c0_gamma, d0_c0_beta, d0_c0_mean, d0_c0_var),
            _flat_lane(d0_c1_w, 256),
            *_affine(d0_c1_gamma, d0_c1_beta, d0_c1_mean, d0_c1_var),
            _flat_lane(d0_c2_w, 256),
            *_affine(d0_c2_gamma, d0_c2_beta, d0_c2_mean, d0_c2_var),
            _flat(d1_c0_w),
            *_affine(d1_c0_gamma, d1_c0_beta, d1_c0_mean, d1_c0_var),
            _flat_lane(d1_c1_w, 384),
            *_affine(d1_c1_gamma, d1_c1_beta, d1_c1_mean, d1_c1_var),
            _flat_lane(d1_c2_w, 384),
            *_affine(d1_c2_gamma, d1_c2_beta, d1_c2_mean, d1_c2_var),
            _flat_lane(u2_c1_w, 384),
            *_affine(u2_c1_gamma, u2_c1_beta, u2_c1_mean, u2_c1_var),
            _flat_lane(u2_c2_w, 384),
            *_affine(u2_c2_gamma, u2_c2_beta, u2_c2_mean, u2_c2_var),
            _flat_lane(_shuffle_cols(u2_c3_w), 384),
            _flat_lane(u1_c1_w, 256),
            *_affine(u1_c1_gamma, u1_c1_beta, u1_c1_mean, u1_c1_var),
            _flat_lane(u1_c2_w, 256),
            *_affine(u1_c2_gamma, u1_c2_beta, u1_c2_mean, u1_c2_var),
            _flat_lane(_shuffle_cols(u1_c3_w), 256),
            _flat_lane(o_c1_w, 128),
            *_affine(o_c1_gamma, o_c1_beta, o_c1_mean, o_c1_var),
            _flat_lane(o_c2_w, 128)]

    in_specs = [pl.BlockSpec((1, H, W, 12), lambda b: (b, 0, 0, 0))]
    for a in args[1:]:
        nd = a.ndim
        in_specs.append(pl.BlockSpec(a.shape, lambda b, _n=nd: (0,) * _n))

    y = pl.pallas_call(
        _body,
        out_shape=jax.ShapeDtypeStruct((B, H, W, 3), jnp.float32),
        grid_spec=pltpu.PrefetchScalarGridSpec(
            num_scalar_prefetch=0,
            grid=(B,),
            in_specs=in_specs,
            out_specs=pl.BlockSpec((1, H, W, 3), lambda b: (b, 0, 0, 0)),
            scratch_shapes=[
                pltpu.VMEM((H + 2, W, 128), _BF),
                pltpu.VMEM((H + 2, W, 384), _BF),
                pltpu.VMEM((H + 2, W, 128), _BF),
                pltpu.VMEM((H // 2 + 2, W // 2, 256), _BF),
                pltpu.VMEM((H // 4 + 2, W // 4, 384), _BF),
                pltpu.VMEM((H + 2, W + 2, 32), jnp.float32),
                pltpu.VMEM((H // 2 + 2, W // 2 + 2, 64), jnp.float32),
                pltpu.VMEM((H // 2, W // 2, 64), jnp.float32),
                pltpu.VMEM((H, W, 32), jnp.float32),
            ]),
        compiler_params=pltpu.CompilerParams(
            dimension_semantics=("parallel",),
            vmem_limit_bytes=100 * 1024 * 1024),
    )(*args)

    return jnp.transpose(nh(in1) - y, (0, 3, 1, 2))


# speedup vs baseline: 2.4965x; 1.9623x over previous
import jax
import jax.numpy as jnp
from jax.experimental import pallas as pl
from jax.experimental.pallas import tpu as pltpu

_EPS = 1e-5
_BF = jnp.bfloat16


def _body(in0_ref, nm_ref, in1_ref, in2_ref, i12_ref,
          w_inc1, s_inc1, b_inc1, w_inc2, s_inc2, b_inc2,
          w_d0c0, s_d0c0, b_d0c0, w_d0c1, s_d0c1, b_d0c1,
          w_d0c2, s_d0c2, b_d0c2,
          w_d1c0, s_d1c0, b_d1c0, w_d1c1, s_d1c1, b_d1c1,
          w_d1c2, s_d1c2, b_d1c2,
          w_u2c1, s_u2c1, b_u2c1, w_u2c2, s_u2c2, b_u2c2, w_u2c3,
          w_u1c1, s_u1c1, b_u1c1, w_u1c2, s_u1c2, b_u1c2, w_u1c3,
          w_oc1, s_oc1, b_oc1, w_oc2,
          o_ref,
          sh12, sh90, sh32, sh64, sh128, pads32, pads64, up2, up1):

    def conv(act, sh, w_ref, sb, relu, out_t=False):
        h2, w, c3 = sh.shape
        hi = h2 - 2
        cin = act.shape[-1]
        a = act.astype(_BF)
        zc = jnp.zeros((hi, 1, cin), _BF)
        pieces = [jnp.concatenate([zc, a[:, :w - 1, :]], axis=1), a,
                  jnp.concatenate([a[:, 1:, :], zc], axis=1)]
        if c3 > 3 * cin:
            pieces.append(jnp.zeros((hi, w, c3 - 3 * cin), _BF))
        zrow = jnp.zeros((1, w, c3), _BF)
        sh[0:1, :, :] = zrow
        sh[h2 - 1:h2, :, :] = zrow
        sh[1:hi + 1, :, :] = jnp.concatenate(pieces, axis=-1)
        slab = jnp.concatenate([sh[pl.ds(dy, hi), :, :] for dy in range(3)],
                               axis=-1).reshape(hi * w, 3 * c3)
        if out_t:
            return jax.lax.dot_general(
                w_ref[...], slab, (((0,), (1,)), ((), ())),
                preferred_element_type=jnp.float32)
        y = jnp.dot(slab, w_ref[...], preferred_element_type=jnp.float32)
        if sb is not None:
            y = y * sb[0][...] + sb[1][...]
        if relu:
            y = jnp.maximum(y, 0.0)
        return y.reshape(hi, w, y.shape[-1])

    def conv_s2(act, pad, w_ref, sb, relu):
        h2, w2, cin = pad.shape
        hi, wi = h2 - 2, w2 - 2
        pad[...] = jnp.zeros_like(pad)
        pad[1:hi + 1, 1:wi + 1, :] = act.astype(pad.dtype)
        ho, wo = hi // 2, wi // 2
        taps = [pad[pl.ds(dy, ho, 2), pl.ds(dx, wo, 2), :]
                for dy in range(3) for dx in range(3)]
        slab = jnp.concatenate(taps, axis=-1).reshape(ho * wo, 9 * cin)
        y = jnp.dot(slab.astype(_BF), w_ref[...],
                    preferred_element_type=jnp.float32)
        if sb is not None:
            y = y * sb[0][...] + sb[1][...]
        if relu:
            y = jnp.maximum(y, 0.0)
        return y.reshape(ho, wo, y.shape[-1])

    def shuffle(y, up_ref):
        hq, wq, c4 = y.shape
        c = c4 // 4
        for r1 in range(2):
            for r2 in range(2):
                q = 2 * r1 + r2
                up_ref[pl.ds(r1, hq, 2), pl.ds(r2, wq, 2), :] = (
                    y[:, :, q * c:(q + 1) * c])

    hh, ww = sh12.shape[0] - 2, sh12.shape[1]
    arr = jnp.concatenate(
        [in0_ref[0], nm_ref[0], in1_ref[0], nm_ref[0], in2_ref[0], nm_ref[0]],
        axis=0).astype(_BF)
    x = jax.lax.dot_general(
        arr, i12_ref[...], (((0,), (0,)), ((), ())),
        preferred_element_type=jnp.float32).astype(_BF).reshape(hh, ww, 12)
    x0 = conv(x, sh12, w_inc1, (s_inc1, b_inc1), True)
    x0 = conv(x0, sh90, w_inc2, (s_inc2, b_inc2), True)
    t = conv_s2(x0, pads32, w_d0c0, (s_d0c0, b_d0c0), True)
    t = conv(t, sh64, w_d0c1, (s_d0c1, b_d0c1), True)
    x1 = conv(t, sh64, w_d0c2, (s_d0c2, b_d0c2), True)
    t = conv_s2(x1, pads64, w_d1c0, (s_d1c0, b_d1c0), True)
    t = conv(t, sh128, w_d1c1, (s_d1c1, b_d1c1), True)
    t = conv(t, sh128, w_d1c2, (s_d1c2, b_d1c2), True)
    t = conv(t, sh128, w_u2c1, (s_u2c1, b_u2c1), True)
    t = conv(t, sh128, w_u2c2, (s_u2c2, b_u2c2), True)
    t = conv(t, sh128, w_u2c3, None, False)
    shuffle(t, up2)
    t = x1 + up2[...]
    t = conv(t, sh64, w_u1c1, (s_u1c1, b_u1c1), True)
    t = conv(t, sh64, w_u1c2, (s_u1c2, b_u1c2), True)
    t = conv(t, sh64, w_u1c3, None, False)
    shuffle(t, up1)
    t = x0 + up1[...]
    t = conv(t, sh32, w_oc1, (s_oc1, b_oc1), True)
    yt = conv(t, sh32, w_oc2, None, False, out_t=True)
    o_ref[...] = (in1_ref[0] - yt)[None]


def _affine(gamma, beta, mean, var):
    s = gamma / jnp.sqrt(var + _EPS)
    return s[None, :].astype(jnp.float32), (beta - mean * s)[None, :].astype(
        jnp.float32)


def _flat(w):
    return w.reshape(9 * w.shape[2], w.shape[3]).astype(_BF)


def _flat_lane(w, c3):
    cin, cout = w.shape[2], w.shape[3]
    wf = w.reshape(3, 3 * cin, cout)
    wp = jnp.zeros((3, c3, cout), w.dtype).at[:, :3 * cin, :].set(wf)
    return wp.reshape(3 * c3, cout).astype(_BF)


def _shuffle_cols(w):
    kh, kw, cin, cout = w.shape
    return (w.reshape(kh, kw, cin, cout // 4, 2, 2)
            .transpose(0, 1, 2, 4, 5, 3).reshape(kh, kw, cin, cout))


def _block_diag_grouped(w, groups):
    kh, kw, cin_g, cout = w.shape
    cin, cout_g = cin_g * groups, cout // groups
    wd = jnp.zeros((kh, kw, cin, cout), w.dtype)
    for g in range(groups):
        wd = wd.at[:, :, g * cin_g:(g + 1) * cin_g,
                   g * cout_g:(g + 1) * cout_g].set(
                       w[:, :, :, g * cout_g:(g + 1) * cout_g])
    return wd


def kernel(in0, in1, in2, noise_map,
           inc1_w, inc1_gamma, inc1_beta, inc1_mean, inc1_var,
           inc2_w, inc2_gamma, inc2_beta, inc2_mean, inc2_var,
           d0_c0_w, d0_c0_gamma, d0_c0_beta, d0_c0_mean, d0_c0_var,
           d0_c1_w, d0_c1_gamma, d0_c1_beta, d0_c1_mean, d0_c1_var,
           d0_c2_w, d0_c2_gamma, d0_c2_beta, d0_c2_mean, d0_c2_var,
           d1_c0_w, d1_c0_gamma, d1_c0_beta, d1_c0_mean, d1_c0_var,
           d1_c1_w, d1_c1_gamma, d1_c1_beta, d1_c1_mean, d1_c1_var,
           d1_c2_w, d1_c2_gamma, d1_c2_beta, d1_c2_mean, d1_c2_var,
           u2_c1_w, u2_c1_gamma, u2_c1_beta, u2_c1_mean, u2_c1_var,
           u2_c2_w, u2_c2_gamma, u2_c2_beta, u2_c2_mean, u2_c2_var,
           u2_c3_w,
           u1_c1_w, u1_c1_gamma, u1_c1_beta, u1_c1_mean, u1_c1_var,
           u1_c2_w, u1_c2_gamma, u1_c2_beta, u1_c2_mean, u1_c2_var,
           u1_c3_w,
           o_c1_w, o_c1_gamma, o_c1_beta, o_c1_mean, o_c1_var,
           o_c2_w):
    B, _, H, W = in0.shape
    HW = H * W
    args = [in0.reshape(B, 3, HW), noise_map.reshape(B, 1, HW),
            in1.reshape(B, 3, HW), in2.reshape(B, 3, HW),
            jnp.eye(12, dtype=_BF),
            _flat_lane(_block_diag_grouped(inc1_w, 3), 128),
            *_affine(inc1_gamma, inc1_beta, inc1_mean, inc1_var),
            _flat_lane(inc2_w, 384),
            *_affine(inc2_gamma, inc2_beta, inc2_mean, inc2_var),
            _flat(d0_c0_w),
            *_affine(d0_c0_gamma, d0_c0_beta, d0_c0_mean, d0_c0_var),
            _flat_lane(d0_c1_w, 256),
            *_affine(d0_c1_gamma, d0_c1_beta, d0_c1_mean, d0_c1_var),
            _flat_lane(d0_c2_w, 256),
            *_affine(d0_c2_gamma, d0_c2_beta, d0_c2_mean, d0_c2_var),
            _flat(d1_c0_w),
            *_affine(d1_c0_gamma, d1_c0_beta, d1_c0_mean, d1_c0_var),
            _flat_lane(d1_c1_w, 384),
            *_affine(d1_c1_gamma, d1_c1_beta, d1_c1_mean, d1_c1_var),
            _flat_lane(d1_c2_w, 384),
            *_affine(d1_c2_gamma, d1_c2_beta, d1_c2_mean, d1_c2_var),
            _flat_lane(u2_c1_w, 384),
            *_affine(u2_c1_gamma, u2_c1_beta, u2_c1_mean, u2_c1_var),
            _flat_lane(u2_c2_w, 384),
            *_affine(u2_c2_gamma, u2_c2_beta, u2_c2_mean, u2_c2_var),
            _flat_lane(_shuffle_cols(u2_c3_w), 384),
            _flat_lane(u1_c1_w, 256),
            *_affine(u1_c1_gamma, u1_c1_beta, u1_c1_mean, u1_c1_var),
            _flat_lane(u1_c2_w, 256),
            *_affine(u1_c2_gamma, u1_c2_beta, u1_c2_mean, u1_c2_var),
            _flat_lane(_shuffle_cols(u1_c3_w), 256),
            _flat_lane(o_c1_w, 128),
            *_affine(o_c1_gamma, o_c1_beta, o_c1_mean, o_c1_var),
            _flat_lane(o_c2_w, 128)]

    in_specs = [pl.BlockSpec((1, 3, HW), lambda b: (b, 0, 0)),
                pl.BlockSpec((1, 1, HW), lambda b: (b, 0, 0)),
                pl.BlockSpec((1, 3, HW), lambda b: (b, 0, 0)),
                pl.BlockSpec((1, 3, HW), lambda b: (b, 0, 0))]
    for a in args[4:]:
        nd = a.ndim
        in_specs.append(pl.BlockSpec(a.shape, lambda b, _n=nd: (0,) * _n))

    y = pl.pallas_call(
        _body,
        out_shape=jax.ShapeDtypeStruct((B, 3, HW), jnp.float32),
        grid_spec=pltpu.PrefetchScalarGridSpec(
            num_scalar_prefetch=0,
            grid=(B,),
            in_specs=in_specs,
            out_specs=pl.BlockSpec((1, 3, HW), lambda b: (b, 0, 0)),
            scratch_shapes=[
                pltpu.VMEM((H + 2, W, 128), _BF),
                pltpu.VMEM((H + 2, W, 384), _BF),
                pltpu.VMEM((H + 2, W, 128), _BF),
                pltpu.VMEM((H // 2 + 2, W // 2, 256), _BF),
                pltpu.VMEM((H // 4 + 2, W // 4, 384), _BF),
                pltpu.VMEM((H + 2, W + 2, 32), jnp.float32),
                pltpu.VMEM((H // 2 + 2, W // 2 + 2, 64), jnp.float32),
                pltpu.VMEM((H // 2, W // 2, 64), jnp.float32),
                pltpu.VMEM((H, W, 32), jnp.float32),
            ]),
        compiler_params=pltpu.CompilerParams(
            dimension_semantics=("parallel",),
            vmem_limit_bytes=100 * 1024 * 1024),
    )(*args)

    return y.reshape(B, 3, H, W)


# speedup vs baseline: 2.4977x; 1.0005x over previous
import jax
import jax.numpy as jnp
from jax.experimental import pallas as pl
from jax.experimental.pallas import tpu as pltpu

_EPS = 1e-5
_BF = jnp.bfloat16


def _body(in0_ref, nm_ref, in1_ref, in2_ref, i12_ref,
          w_inc1, s_inc1, b_inc1, w_inc2, s_inc2, b_inc2,
          w_d0c0, s_d0c0, b_d0c0, w_d0c1, s_d0c1, b_d0c1,
          w_d0c2, s_d0c2, b_d0c2,
          w_d1c0, s_d1c0, b_d1c0, w_d1c1, s_d1c1, b_d1c1,
          w_d1c2, s_d1c2, b_d1c2,
          w_u2c1, s_u2c1, b_u2c1, w_u2c2, s_u2c2, b_u2c2, w_u2c3,
          w_u1c1, s_u1c1, b_u1c1, w_u1c2, s_u1c2, b_u1c2, w_u1c3,
          w_oc1, s_oc1, b_oc1, w_oc2,
          o_ref,
          sh12, sh90, sh32, sh64, sh128, pads32, pads64, up2, up1):

    def conv(act, sh, w_ref, sb, relu, out_t=False):
        h2, w, c3 = sh.shape
        hi = h2 - 2
        cin = act.shape[-1]
        a = act.astype(_BF)
        zc = jnp.zeros((hi, 1, cin), _BF)
        pieces = [jnp.concatenate([zc, a[:, :w - 1, :]], axis=1), a,
                  jnp.concatenate([a[:, 1:, :], zc], axis=1)]
        if c3 > 3 * cin:
            pieces.append(jnp.zeros((hi, w, c3 - 3 * cin), _BF))
        zrow = jnp.zeros((1, w, c3), _BF)
        sh[0:1, :, :] = zrow
        sh[h2 - 1:h2, :, :] = zrow
        sh[1:hi + 1, :, :] = jnp.concatenate(pieces, axis=-1)
        slab = jnp.concatenate([sh[pl.ds(dy, hi), :, :] for dy in range(3)],
                               axis=-1).reshape(hi * w, 3 * c3)
        if out_t:
            return jax.lax.dot_general(
                w_ref[...], slab, (((0,), (1,)), ((), ())),
                preferred_element_type=jnp.float32)
        y = jnp.dot(slab, w_ref[...], preferred_element_type=jnp.float32)
        if sb is not None:
            y = y * sb[0][...] + sb[1][...]
        if relu:
            y = jnp.maximum(y, 0.0)
        return y.reshape(hi, w, y.shape[-1])

    def conv_s2(act, pad, w_ref, sb, relu):
        h2, w2, cin = pad.shape
        hi, wi = h2 - 2, w2 - 2
        pad[...] = jnp.zeros_like(pad)
        pad[1:hi + 1, 1:wi + 1, :] = act.astype(pad.dtype)
        ho, wo = hi // 2, wi // 2
        taps = [pad[pl.ds(dy, ho, 2), pl.ds(dx, wo, 2), :]
                for dy in range(3) for dx in range(3)]
        slab = jnp.concatenate(taps, axis=-1).reshape(ho * wo, 9 * cin)
        y = jnp.dot(slab.astype(_BF), w_ref[...],
                    preferred_element_type=jnp.float32)
        if sb is not None:
            y = y * sb[0][...] + sb[1][...]
        if relu:
            y = jnp.maximum(y, 0.0)
        return y.reshape(ho, wo, y.shape[-1])

    def shuffle(y, up_ref):
        hq, wq, c4 = y.shape
        c = c4 // 4
        for r1 in range(2):
            for r2 in range(2):
                q = 2 * r1 + r2
                up_ref[pl.ds(r1, hq, 2), pl.ds(r2, wq, 2), :] = (
                    y[:, :, q * c:(q + 1) * c])

    hh, ww = sh12.shape[1] - 2, sh12.shape[2]
    for im in range(2):
        arr = jnp.concatenate(
            [in0_ref[im], nm_ref[im], in1_ref[im], nm_ref[im], in2_ref[im],
             nm_ref[im]], axis=0).astype(_BF)
        x = jax.lax.dot_general(
            arr, i12_ref[...], (((0,), (0,)), ((), ())),
            preferred_element_type=jnp.float32).astype(_BF).reshape(hh, ww,
                                                                    12)
        x0 = conv(x, sh12.at[im], w_inc1, (s_inc1, b_inc1), True)
        x0 = conv(x0, sh90.at[im], w_inc2, (s_inc2, b_inc2), True)
        t = conv_s2(x0, pads32.at[im], w_d0c0, (s_d0c0, b_d0c0), True)
        t = conv(t, sh64.at[im], w_d0c1, (s_d0c1, b_d0c1), True)
        x1 = conv(t, sh64.at[im], w_d0c2, (s_d0c2, b_d0c2), True)
        t = conv_s2(x1, pads64.at[im], w_d1c0, (s_d1c0, b_d1c0), True)
        t = conv(t, sh128.at[im], w_d1c1, (s_d1c1, b_d1c1), True)
        t = conv(t, sh128.at[im], w_d1c2, (s_d1c2, b_d1c2), True)
        t = conv(t, sh128.at[im], w_u2c1, (s_u2c1, b_u2c1), True)
        t = conv(t, sh128.at[im], w_u2c2, (s_u2c2, b_u2c2), True)
        t = conv(t, sh128.at[im], w_u2c3, None, False)
        shuffle(t, up2.at[im])
        t = x1 + up2[im]
        t = conv(t, sh64.at[im], w_u1c1, (s_u1c1, b_u1c1), True)
        t = conv(t, sh64.at[im], w_u1c2, (s_u1c2, b_u1c2), True)
        t = conv(t, sh64.at[im], w_u1c3, None, False)
        shuffle(t, up1.at[im])
        t = x0 + up1[im]
        t = conv(t, sh32.at[im], w_oc1, (s_oc1, b_oc1), True)
        yt = conv(t, sh32.at[im], w_oc2, None, False, out_t=True)
        o_ref[im, :, :] = in1_ref[im] - yt


def _affine(gamma, beta, mean, var):
    s = gamma / jnp.sqrt(var + _EPS)
    return s[None, :].astype(jnp.float32), (beta - mean * s)[None, :].astype(
        jnp.float32)


def _flat(w):
    return w.reshape(9 * w.shape[2], w.shape[3]).astype(_BF)


def _flat_lane(w, c3):
    cin, cout = w.shape[2], w.shape[3]
    wf = w.reshape(3, 3 * cin, cout)
    wp = jnp.zeros((3, c3, cout), w.dtype).at[:, :3 * cin, :].set(wf)
    return wp.reshape(3 * c3, cout).astype(_BF)


def _shuffle_cols(w):
    kh, kw, cin, cout = w.shape
    return (w.reshape(kh, kw, cin, cout // 4, 2, 2)
            .transpose(0, 1, 2, 4, 5, 3).reshape(kh, kw, cin, cout))


def _block_diag_grouped(w, groups):
    kh, kw, cin_g, cout = w.shape
    cin, cout_g = cin_g * groups, cout // groups
    wd = jnp.zeros((kh, kw, cin, cout), w.dtype)
    for g in range(groups):
        wd = wd.at[:, :, g * cin_g:(g + 1) * cin_g,
                   g * cout_g:(g + 1) * cout_g].set(
                       w[:, :, :, g * cout_g:(g + 1) * cout_g])
    return wd


def kernel(in0, in1, in2, noise_map,
           inc1_w, inc1_gamma, inc1_beta, inc1_mean, inc1_var,
           inc2_w, inc2_gamma, inc2_beta, inc2_mean, inc2_var,
           d0_c0_w, d0_c0_gamma, d0_c0_beta, d0_c0_mean, d0_c0_var,
           d0_c1_w, d0_c1_gamma, d0_c1_beta, d0_c1_mean, d0_c1_var,
           d0_c2_w, d0_c2_gamma, d0_c2_beta, d0_c2_mean, d0_c2_var,
           d1_c0_w, d1_c0_gamma, d1_c0_beta, d1_c0_mean, d1_c0_var,
           d1_c1_w, d1_c1_gamma, d1_c1_beta, d1_c1_mean, d1_c1_var,
           d1_c2_w, d1_c2_gamma, d1_c2_beta, d1_c2_mean, d1_c2_var,
           u2_c1_w, u2_c1_gamma, u2_c1_beta, u2_c1_mean, u2_c1_var,
           u2_c2_w, u2_c2_gamma, u2_c2_beta, u2_c2_mean, u2_c2_var,
           u2_c3_w,
           u1_c1_w, u1_c1_gamma, u1_c1_beta, u1_c1_mean, u1_c1_var,
           u1_c2_w, u1_c2_gamma, u1_c2_beta, u1_c2_mean, u1_c2_var,
           u1_c3_w,
           o_c1_w, o_c1_gamma, o_c1_beta, o_c1_mean, o_c1_var,
           o_c2_w):
    B, _, H, W = in0.shape
    HW = H * W
    args = [in0.reshape(B, 3, HW), noise_map.reshape(B, 1, HW),
            in1.reshape(B, 3, HW), in2.reshape(B, 3, HW),
            jnp.eye(12, dtype=_BF),
            _flat_lane(_block_diag_grouped(inc1_w, 3), 128),
            *_affine(inc1_gamma, inc1_beta, inc1_mean, inc1_var),
            _flat_lane(inc2_w, 384),
            *_affine(inc2_gamma, inc2_beta, inc2_mean, inc2_var),
            _flat(d0_c0_w),
            *_affine(d0_c0_gamma, d0_c0_beta, d0_c0_mean, d0_c0_var),
            _flat_lane(d0_c1_w, 256),
            *_affine(d0_c1_gamma, d0_c1_beta, d0_c1_mean, d0_c1_var),
            _flat_lane(d0_c2_w, 256),
            *_affine(d0_c2_gamma, d0_c2_beta, d0_c2_mean, d0_c2_var),
            _flat(d1_c0_w),
            *_affine(d1_c0_gamma, d1_c0_beta, d1_c0_mean, d1_c0_var),
            _flat_lane(d1_c1_w, 384),
            *_affine(d1_c1_gamma, d1_c1_beta, d1_c1_mean, d1_c1_var),
            _flat_lane(d1_c2_w, 384),
            *_affine(d1_c2_gamma, d1_c2_beta, d1_c2_mean, d1_c2_var),
            _flat_lane(u2_c1_w, 384),
            *_affine(u2_c1_gamma, u2_c1_beta, u2_c1_mean, u2_c1_var),
            _flat_lane(u2_c2_w, 384),
            *_affine(u2_c2_gamma, u2_c2_beta, u2_c2_mean, u2_c2_var),
            _flat_lane(_shuffle_cols(u2_c3_w), 384),
            _flat_lane(u1_c1_w, 256),
            *_affine(u1_c1_gamma, u1_c1_beta, u1_c1_mean, u1_c1_var),
            _flat_lane(u1_c2_w, 256),
            *_affine(u1_c2_gamma, u1_c2_beta, u1_c2_mean, u1_c2_var),
            _flat_lane(_shuffle_cols(u1_c3_w), 256),
            _flat_lane(o_c1_w, 128),
            *_affine(o_c1_gamma, o_c1_beta, o_c1_mean, o_c1_var),
            _flat_lane(o_c2_w, 128)]

    in_specs = [pl.BlockSpec((2, 3, HW), lambda b: (b, 0, 0)),
                pl.BlockSpec((2, 1, HW), lambda b: (b, 0, 0)),
                pl.BlockSpec((2, 3, HW), lambda b: (b, 0, 0)),
                pl.BlockSpec((2, 3, HW), lambda b: (b, 0, 0))]
    for a in args[4:]:
        nd = a.ndim
        in_specs.append(pl.BlockSpec(a.shape, lambda b, _n=nd: (0,) * _n))

    y = pl.pallas_call(
        _body,
        out_shape=jax.ShapeDtypeStruct((B, 3, HW), jnp.float32),
        grid_spec=pltpu.PrefetchScalarGridSpec(
            num_scalar_prefetch=0,
            grid=(B // 2,),
            in_specs=in_specs,
            out_specs=pl.BlockSpec((2, 3, HW), lambda b: (b, 0, 0)),
            scratch_shapes=[
                pltpu.VMEM((2, H + 2, W, 128), _BF),
                pltpu.VMEM((2, H + 2, W, 384), _BF),
                pltpu.VMEM((2, H + 2, W, 128), _BF),
                pltpu.VMEM((2, H // 2 + 2, W // 2, 256), _BF),
                pltpu.VMEM((2, H // 4 + 2, W // 4, 384), _BF),
                pltpu.VMEM((2, H + 2, W + 2, 32), jnp.float32),
                pltpu.VMEM((2, H // 2 + 2, W // 2 + 2, 64), jnp.float32),
                pltpu.VMEM((2, H // 2, W // 2, 64), jnp.float32),
                pltpu.VMEM((2, H, W, 32), jnp.float32),
            ]),
        compiler_params=pltpu.CompilerParams(
            dimension_semantics=("parallel",),
            vmem_limit_bytes=100 * 1024 * 1024),
    )(*args)

    return y.reshape(B, 3, H, W)


# speedup vs baseline: 2.4979x; 1.0001x over previous
import jax
import jax.numpy as jnp
from jax.experimental import pallas as pl
from jax.experimental.pallas import tpu as pltpu

_EPS = 1e-5
_BF = jnp.bfloat16


def _body(in0_ref, nm_ref, in1_ref, in2_ref, i12_ref,
          w_inc1, s_inc1, b_inc1, w_inc2, s_inc2, b_inc2,
          w_d0c0, s_d0c0, b_d0c0, w_d0c1, s_d0c1, b_d0c1,
          w_d0c2, s_d0c2, b_d0c2,
          w_d1c0, s_d1c0, b_d1c0, w_d1c1, s_d1c1, b_d1c1,
          w_d1c2, s_d1c2, b_d1c2,
          w_u2c1, s_u2c1, b_u2c1, w_u2c2, s_u2c2, b_u2c2, w_u2c3,
          w_u1c1, s_u1c1, b_u1c1, w_u1c2, s_u1c2, b_u1c2, w_u1c3,
          w_oc1, s_oc1, b_oc1, w_oc2,
          o_ref,
          sh12, sh90, sh32, sh64, sh128, pads32, pads64, up2, up1):

    def conv(act, sh, w_ref, sb, relu, out_t=False):
        h2, w, c3 = sh.shape
        hi = h2 - 2
        cin = act.shape[-1]
        a = act.astype(_BF)
        zc = jnp.zeros((hi, 1, cin), _BF)
        pieces = [jnp.concatenate([zc, a[:, :w - 1, :]], axis=1), a,
                  jnp.concatenate([a[:, 1:, :], zc], axis=1)]
        if c3 > 3 * cin:
            pieces.append(jnp.zeros((hi, w, c3 - 3 * cin), _BF))
        zrow = jnp.zeros((1, w, c3), _BF)
        sh[0:1, :, :] = zrow
        sh[h2 - 1:h2, :, :] = zrow
        sh[1:hi + 1, :, :] = jnp.concatenate(pieces, axis=-1)
        slab = jnp.concatenate([sh[pl.ds(dy, hi), :, :] for dy in range(3)],
                               axis=-1).reshape(hi * w, 3 * c3)
        if out_t:
            return jax.lax.dot_general(
                w_ref[...], slab, (((0,), (1,)), ((), ())),
                preferred_element_type=jnp.float32)
        y = jnp.dot(slab, w_ref[...], preferred_element_type=jnp.float32)
        if sb is not None:
            y = y * sb[0][...] + sb[1][...]
        if relu:
            y = jnp.maximum(y, 0.0)
        return y.reshape(hi, w, y.shape[-1])

    def conv_s2(act, pad, w_ref, sb, relu):
        h2, w2, cin = pad.shape
        hi, wi = h2 - 2, w2 - 2
        pad[...] = jnp.zeros_like(pad)
        pad[1:hi + 1, 1:wi + 1, :] = act.astype(pad.dtype)
        ho, wo = hi // 2, wi // 2
        taps = [pad[pl.ds(dy, ho, 2), pl.ds(dx, wo, 2), :]
                for dy in range(3) for dx in range(3)]
        slab = jnp.concatenate(taps, axis=-1).reshape(ho * wo, 9 * cin)
        y = jnp.dot(slab.astype(_BF), w_ref[...],
                    preferred_element_type=jnp.float32)
        if sb is not None:
            y = y * sb[0][...] + sb[1][...]
        if relu:
            y = jnp.maximum(y, 0.0)
        return y.reshape(ho, wo, y.shape[-1])

    def shuffle(y, up_ref):
        hq, wq, c4 = y.shape
        c = c4 // 4
        for r1 in range(2):
            for r2 in range(2):
                q = 2 * r1 + r2
                up_ref[pl.ds(r1, hq, 2), pl.ds(r2, wq, 2), :] = (
                    y[:, :, q * c:(q + 1) * c])

    hh, ww = sh12.shape[0] - 2, sh12.shape[1]
    arr = jnp.concatenate(
        [in0_ref[0], nm_ref[0], in1_ref[0], nm_ref[0], in2_ref[0], nm_ref[0]],
        axis=0).astype(_BF)
    x = jax.lax.dot_general(
        arr, i12_ref[...], (((0,), (0,)), ((), ())),
        preferred_element_type=jnp.float32).astype(_BF).reshape(hh, ww, 12)
    x0 = conv(x, sh12, w_inc1, (s_inc1, b_inc1), True)
    x0 = conv(x0, sh90, w_inc2, (s_inc2, b_inc2), True)
    t = conv_s2(x0, pads32, w_d0c0, (s_d0c0, b_d0c0), True)
    t = conv(t, sh64, w_d0c1, (s_d0c1, b_d0c1), True)
    x1 = conv(t, sh64, w_d0c2, (s_d0c2, b_d0c2), True)
    t = conv_s2(x1, pads64, w_d1c0, (s_d1c0, b_d1c0), True)
    t = conv(t, sh128, w_d1c1, (s_d1c1, b_d1c1), True)
    t = conv(t, sh128, w_d1c2, (s_d1c2, b_d1c2), True)
    t = conv(t, sh128, w_u2c1, (s_u2c1, b_u2c1), True)
    t = conv(t, sh128, w_u2c2, (s_u2c2, b_u2c2), True)
    t = conv(t, sh128, w_u2c3, None, False)
    shuffle(t, up2)
    t = x1 + up2[...]
    t = conv(t, sh64, w_u1c1, (s_u1c1, b_u1c1), True)
    t = conv(t, sh64, w_u1c2, (s_u1c2, b_u1c2), True)
    t = conv(t, sh64, w_u1c3, None, False)
    shuffle(t, up1)
    t = x0 + up1[...]
    t = conv(t, sh32, w_oc1, (s_oc1, b_oc1), True)
    yt = conv(t, sh32, w_oc2, None, False, out_t=True)
    o_ref[...] = (in1_ref[0] - yt)[None]


def _affine(gamma, beta, mean, var):
    s = gamma / jnp.sqrt(var + _EPS)
    return s[None, :].astype(jnp.float32), (beta - mean * s)[None, :].astype(
        jnp.float32)


def _flat(w):
    return w.reshape(9 * w.shape[2], w.shape[3]).astype(_BF)


def _flat_lane(w, c3):
    cin, cout = w.shape[2], w.shape[3]
    wf = w.reshape(3, 3 * cin, cout)
    wp = jnp.zeros((3, c3, cout), w.dtype).at[:, :3 * cin, :].set(wf)
    return wp.reshape(3 * c3, cout).astype(_BF)


def _shuffle_cols(w):
    kh, kw, cin, cout = w.shape
    return (w.reshape(kh, kw, cin, cout // 4, 2, 2)
            .transpose(0, 1, 2, 4, 5, 3).reshape(kh, kw, cin, cout))


def _block_diag_grouped(w, groups):
    kh, kw, cin_g, cout = w.shape
    cin, cout_g = cin_g * groups, cout // groups
    wd = jnp.zeros((kh, kw, cin, cout), w.dtype)
    for g in range(groups):
        wd = wd.at[:, :, g * cin_g:(g + 1) * cin_g,
                   g * cout_g:(g + 1) * cout_g].set(
                       w[:, :, :, g * cout_g:(g + 1) * cout_g])
    return wd


def kernel(in0, in1, in2, noise_map,
           inc1_w, inc1_gamma, inc1_beta, inc1_mean, inc1_var,
           inc2_w, inc2_gamma, inc2_beta, inc2_mean, inc2_var,
           d0_c0_w, d0_c0_gamma, d0_c0_beta, d0_c0_mean, d0_c0_var,
           d0_c1_w, d0_c1_gamma, d0_c1_beta, d0_c1_mean, d0_c1_var,
           d0_c2_w, d0_c2_gamma, d0_c2_beta, d0_c2_mean, d0_c2_var,
           d1_c0_w, d1_c0_gamma, d1_c0_beta, d1_c0_mean, d1_c0_var,
           d1_c1_w, d1_c1_gamma, d1_c1_beta, d1_c1_mean, d1_c1_var,
           d1_c2_w, d1_c2_gamma, d1_c2_beta, d1_c2_mean, d1_c2_var,
           u2_c1_w, u2_c1_gamma, u2_c1_beta, u2_c1_mean, u2_c1_var,
           u2_c2_w, u2_c2_gamma, u2_c2_beta, u2_c2_mean, u2_c2_var,
           u2_c3_w,
           u1_c1_w, u1_c1_gamma, u1_c1_beta, u1_c1_mean, u1_c1_var,
           u1_c2_w, u1_c2_gamma, u1_c2_beta, u1_c2_mean, u1_c2_var,
           u1_c3_w,
           o_c1_w, o_c1_gamma, o_c1_beta, o_c1_mean, o_c1_var,
           o_c2_w):
    B, _, H, W = in0.shape
    HW = H * W
    args = [in0.reshape(B, 3, HW), noise_map.reshape(B, 1, HW),
            in1.reshape(B, 3, HW), in2.reshape(B, 3, HW),
            jnp.eye(12, dtype=_BF),
            _flat_lane(_block_diag_grouped(inc1_w, 3), 128),
            *_affine(inc1_gamma, inc1_beta, inc1_mean, inc1_var),
            _flat_lane(inc2_w, 384),
            *_affine(inc2_gamma, inc2_beta, inc2_mean, inc2_var),
            _flat(d0_c0_w),
            *_affine(d0_c0_gamma, d0_c0_beta, d0_c0_mean, d0_c0_var),
            _flat_lane(d0_c1_w, 256),
            *_affine(d0_c1_gamma, d0_c1_beta, d0_c1_mean, d0_c1_var),
            _flat_lane(d0_c2_w, 256),
            *_affine(d0_c2_gamma, d0_c2_beta, d0_c2_mean, d0_c2_var),
            _flat(d1_c0_w),
            *_affine(d1_c0_gamma, d1_c0_beta, d1_c0_mean, d1_c0_var),
            _flat_lane(d1_c1_w, 384),
            *_affine(d1_c1_gamma, d1_c1_beta, d1_c1_mean, d1_c1_var),
            _flat_lane(d1_c2_w, 384),
            *_affine(d1_c2_gamma, d1_c2_beta, d1_c2_mean, d1_c2_var),
            _flat_lane(u2_c1_w, 384),
            *_affine(u2_c1_gamma, u2_c1_beta, u2_c1_mean, u2_c1_var),
            _flat_lane(u2_c2_w, 384),
            *_affine(u2_c2_gamma, u2_c2_beta, u2_c2_mean, u2_c2_var),
            _flat_lane(_shuffle_cols(u2_c3_w), 384),
            _flat_lane(u1_c1_w, 256),
            *_affine(u1_c1_gamma, u1_c1_beta, u1_c1_mean, u1_c1_var),
            _flat_lane(u1_c2_w, 256),
            *_affine(u1_c2_gamma, u1_c2_beta, u1_c2_mean, u1_c2_var),
            _flat_lane(_shuffle_cols(u1_c3_w), 256),
            _flat_lane(o_c1_w, 128),
            *_affine(o_c1_gamma, o_c1_beta, o_c1_mean, o_c1_var),
            _flat_lane(o_c2_w, 128)]

    in_specs = [pl.BlockSpec((1, 3, HW), lambda b: (b, 0, 0)),
                pl.BlockSpec((1, 1, HW), lambda b: (b, 0, 0)),
                pl.BlockSpec((1, 3, HW), lambda b: (b, 0, 0)),
                pl.BlockSpec((1, 3, HW), lambda b: (b, 0, 0))]
    for a in args[4:]:
        nd = a.ndim
        in_specs.append(pl.BlockSpec(a.shape, lambda b, _n=nd: (0,) * _n))

    y = pl.pallas_call(
        _body,
        out_shape=jax.ShapeDtypeStruct((B, 3, HW), jnp.float32),
        grid_spec=pltpu.PrefetchScalarGridSpec(
            num_scalar_prefetch=0,
            grid=(B,),
            in_specs=in_specs,
            out_specs=pl.BlockSpec((1, 3, HW), lambda b: (b, 0, 0)),
            scratch_shapes=[
                pltpu.VMEM((H + 2, W, 128), _BF),
                pltpu.VMEM((H + 2, W, 384), _BF),
                pltpu.VMEM((H + 2, W, 128), _BF),
                pltpu.VMEM((H // 2 + 2, W // 2, 256), _BF),
                pltpu.VMEM((H // 4 + 2, W // 4, 384), _BF),
                pltpu.VMEM((H + 2, W + 2, 32), jnp.float32),
                pltpu.VMEM((H // 2 + 2, W // 2 + 2, 64), jnp.float32),
                pltpu.VMEM((H // 2, W // 2, 64), jnp.float32),
                pltpu.VMEM((H, W, 32), jnp.float32),
            ]),
        compiler_params=pltpu.CompilerParams(
            dimension_semantics=("parallel",),
            vmem_limit_bytes=100 * 1024 * 1024),
    )(*args)

    return y.reshape(B, 3, H, W)
